# Initial kernel scaffold; baseline (speedup 1.0000x reference)
import jax
import jax.numpy as jnp
from jax.experimental import pallas as pl


def kernel(x, edge_index, batch, W1, b1, W2, b2, W3, b3, Wc, bc, Wl1, bl1, Wl2, bl2):
    raise NotImplementedError("write your pallas kernel here")


# speedup vs baseline: 9.9064x; 9.9064x over previous
import functools

import jax
import jax.numpy as jnp
from jax import lax
from jax.experimental import pallas as pl
from jax.experimental.pallas import tpu as pltpu
from jax.experimental.pallas import tpu_sc as plsc

N_NODES = 10000
DIM = 128
NUM_GRAPHS = 128
TOPK = 30
CONV_OUT = 32
KSIZE = 5
NUM_CLASSES = 10

NPAD = 10240
ZROW = NPAD - 1
NCORES = 2
NSUB = 16
NWORK = NCORES * NSUB
CHUNK = 128

def _mk_mesh():
  return plsc.VectorSubcoreMesh(core_axis_name="c", subcore_axis_name="s",
                                num_cores=NCORES, num_subcores=NSUB)


def _wid():
  return lax.axis_index("s") * NCORES + lax.axis_index("c")


DEGW = 16


def _make_deg_kernel(epad):
  per_tile = epad // NWORK
  chunks = per_tile // CHUNK
  rpt = NPAD // NSUB

  @functools.partial(
      pl.kernel,
      mesh=_mk_mesh(),
      out_type=jax.ShapeDtypeStruct((NCORES, NPAD, DEGW), jnp.float32),
      scratch_types=[
          pltpu.VMEM((CHUNK,), jnp.int32),
          pltpu.VMEM((CHUNK, DEGW), jnp.float32),
          pltpu.VMEM((CHUNK, DEGW), jnp.float32),
          pltpu.VMEM_SHARED((NPAD, DEGW), jnp.float32),
      ],
  )
  def deg_kernel(dst_hbm, out_hbm, didx_v, ones_v, zrows_v, acc_sh):
    wid = _wid()
    core = lax.axis_index("c")
    sid = lax.axis_index("s")
    zero16 = jnp.zeros((16,), jnp.float32)
    e0 = jnp.where(lax.iota(jnp.int32, 16) == 0, 1.0, 0.0)

    def init_rows(r, c):
      ones_v[r, pl.ds(0, 16)] = e0
      zrows_v[r, pl.ds(0, 16)] = zero16
      return c
    lax.fori_loop(0, CHUNK, init_rows, 0)

    def zacc(i, c):
      pltpu.sync_copy(zrows_v, acc_sh.at[pl.ds(sid * rpt + i * CHUNK, CHUNK)])
      return c
    lax.fori_loop(0, rpt // CHUNK, zacc, 0)
    plsc.subcore_barrier()

    def chunk_body(ci, c):
      base = wid * per_tile + ci * CHUNK
      pltpu.sync_copy(dst_hbm.at[pl.ds(base, CHUNK)], didx_v)
      pltpu.sync_copy(ones_v, acc_sh.at[didx_v], add=True)
      return c
    lax.fori_loop(0, chunks, chunk_body, 0)
    plsc.subcore_barrier()

    pltpu.sync_copy(acc_sh.at[pl.ds(sid * rpt, rpt)],
                    out_hbm.at[core, pl.ds(sid * rpt, rpt)])

  return deg_kernel


def _make_agg_kernel(epad):
  per_tile = epad // NWORK
  chunks = per_tile // CHUNK
  rpt = NPAD // NSUB

  @functools.partial(
      pl.kernel,
      mesh=_mk_mesh(),
      out_type=jax.ShapeDtypeStruct((NCORES, NPAD, DIM), jnp.float32),
      scratch_types=[
          pltpu.VMEM((CHUNK,), jnp.int32),
          pltpu.VMEM((CHUNK,), jnp.int32),
          pltpu.VMEM((CHUNK, DIM), jnp.float32),
          pltpu.VMEM((CHUNK, DIM), jnp.float32),
          pltpu.VMEM_SHARED((NPAD, DIM), jnp.float32),
          pltpu.SemaphoreType.DMA,
      ],
  )
  def agg_kernel(hs_hbm, src_hbm, dst_hbm, out_hbm,
                 sidx_v, didx_v, rows_v, zero_v, acc_sh, sem):
    wid = _wid()
    core = lax.axis_index("c")
    sid = lax.axis_index("s")
    zero16 = jnp.zeros((16,), jnp.float32)

    def zrow(r, c):
      def zcol(j, cc):
        zero_v[r, pl.ds(j * 16, 16)] = zero16
        return cc
      lax.fori_loop(0, DIM // 16, zcol, 0)
      return c
    lax.fori_loop(0, CHUNK, zrow, 0)

    def zacc(i, c):
      pltpu.sync_copy(zero_v, acc_sh.at[pl.ds(sid * rpt + i * CHUNK, CHUNK)])
      return c
    lax.fori_loop(0, rpt // CHUNK, zacc, 0)
    plsc.subcore_barrier()

    def chunk_body(ci, c):
      base = wid * per_tile + ci * CHUNK
      pltpu.sync_copy(src_hbm.at[pl.ds(base, CHUNK)], sidx_v)
      pltpu.sync_copy(dst_hbm.at[pl.ds(base, CHUNK)], didx_v)
      pltpu.async_copy(hs_hbm.at[sidx_v], rows_v, sem).wait()
      pltpu.sync_copy(rows_v, acc_sh.at[didx_v], add=True)
      return c
    lax.fori_loop(0, chunks, chunk_body, 0)
    plsc.subcore_barrier()

    pltpu.sync_copy(acc_sh.at[pl.ds(sid * rpt, rpt)],
                    out_hbm.at[core, pl.ds(sid * rpt, rpt)])

  return agg_kernel


def _make_row_gather_kernel(nrows):
  per_tile = nrows // NWORK

  @functools.partial(
      pl.kernel,
      mesh=_mk_mesh(),
      out_type=jax.ShapeDtypeStruct((nrows, DIM), jnp.float32),
      scratch_types=[
          pltpu.VMEM((per_tile,), jnp.int32),
          pltpu.VMEM((per_tile, DIM), jnp.float32),
          pltpu.SemaphoreType.DMA,
      ],
  )
  def gather_kernel(table_hbm, idx_hbm, out_hbm, idx_v, rows_v, sem):
    wid = _wid()
    base = wid * per_tile
    pltpu.sync_copy(idx_hbm.at[pl.ds(base, per_tile)], idx_v)
    pltpu.async_copy(table_hbm.at[idx_v], rows_v, sem).wait()
    pltpu.sync_copy(rows_v, out_hbm.at[pl.ds(base, per_tile)])

  return gather_kernel


_ROWS_BLK = 2048
_N_BLKS = NPAD // _ROWS_BLK


def _dis_col(pc):
  deg = pc[:, 0:1] + pc[:, 1:2] + 1.0
  return lax.rsqrt(deg)


def _t0_body(x_ref, pc_ref, w_ref, o_ref):
  dis = _dis_col(pc_ref[...])
  o_ref[...] = dis * jnp.dot(x_ref[...], w_ref[...],
                             preferred_element_type=jnp.float32)


def _tmid_body(a0_ref, a1_ref, hs_ref, pc_ref, b_ref, w_ref, o_ref):
  dis = _dis_col(pc_ref[...])
  h = jax.nn.relu(dis * (a0_ref[...] + a1_ref[...] + hs_ref[...])
                  + b_ref[0:1, :])
  o_ref[...] = dis * jnp.dot(h, w_ref[...],
                             preferred_element_type=jnp.float32)


def _tlast_body(a0_ref, a1_ref, hs_ref, pc_ref, b_ref, o_ref):
  dis = _dis_col(pc_ref[...])
  o_ref[...] = jax.nn.relu(dis * (a0_ref[...] + a1_ref[...] + hs_ref[...])
                           + b_ref[0:1, :])


def _rows_spec(width):
  return pl.BlockSpec((_ROWS_BLK, width), lambda i: (i, 0))


def _full_spec(shape):
  return pl.BlockSpec(shape, lambda i: tuple(0 for _ in shape))


def _t0(x, pc, w):
  return pl.pallas_call(
      _t0_body,
      grid=(_N_BLKS,),
      in_specs=[_rows_spec(DIM), _rows_spec(8), _full_spec((DIM, DIM))],
      out_specs=_rows_spec(DIM),
      out_shape=jax.ShapeDtypeStruct((NPAD, DIM), jnp.float32),
  )(x, pc, w)


def _tmid(a0, a1, hs, pc, b8, w):
  return pl.pallas_call(
      _tmid_body,
      grid=(_N_BLKS,),
      in_specs=[_rows_spec(DIM), _rows_spec(DIM), _rows_spec(DIM),
                _rows_spec(8), _full_spec((8, DIM)), _full_spec((DIM, DIM))],
      out_specs=_rows_spec(DIM),
      out_shape=jax.ShapeDtypeStruct((NPAD, DIM), jnp.float32),
  )(a0, a1, hs, pc, b8, w)


def _tlast(a0, a1, hs, pc, b8):
  return pl.pallas_call(
      _tlast_body,
      grid=(_N_BLKS,),
      in_specs=[_rows_spec(DIM), _rows_spec(DIM), _rows_spec(DIM),
                _rows_spec(8), _full_spec((8, DIM))],
      out_specs=_rows_spec(DIM),
      out_shape=jax.ShapeDtypeStruct((NPAD, DIM), jnp.float32),
  )(a0, a1, hs, pc, b8)


def _topk_body(keys_ref, batch_ref, o_ref, km_ref):
  b_iota = lax.broadcasted_iota(jnp.int32, (NPAD, NUM_GRAPHS), 1)
  i_iota = lax.broadcasted_iota(jnp.int32, (NPAD, NUM_GRAPHS), 0)
  keys = keys_ref[:, 7:8]
  batch = batch_ref[:, 7:8]
  km_ref[...] = jnp.where(batch == b_iota, keys, -1e30)

  def body(k, c):
    v = km_ref[...]
    m = jnp.max(v, axis=0, keepdims=True)
    cand = jnp.where(v == m, i_iota, jnp.int32(2**30))
    sel = jnp.min(cand, axis=0, keepdims=True)
    valid = m > -1e20
    o_ref[pl.ds(k, 1), :] = jnp.where(valid, sel, jnp.int32(ZROW))
    km_ref[...] = jnp.where(i_iota == sel, -1e30, v)
    return c
  lax.fori_loop(0, 32, body, 0)


def _topk(keys8, batch8):
  return pl.pallas_call(
      _topk_body,
      out_shape=jax.ShapeDtypeStruct((32, NUM_GRAPHS), jnp.int32),
      scratch_shapes=[pltpu.VMEM((NPAD, NUM_GRAPHS), jnp.float32)],
  )(keys8, batch8)


_CONV_T = TOPK - KSIZE + 1


def _head_body(pf_ref, wc_ref, bc_ref, wl1_ref, bl1_ref, wl2_ref, bl2_ref,
               o_ref):
  qs = []
  for t in range(_CONV_T):
    win = pf_ref[:, DIM * t:DIM * t + DIM * KSIZE]
    qs.append(jax.nn.relu(
        jnp.dot(win, wc_ref[...], preferred_element_type=jnp.float32)
        + bc_ref[0:1, :]))
  q = jnp.concatenate(qs, axis=1)
  z = jax.nn.relu(
      jnp.dot(q, wl1_ref[...], preferred_element_type=jnp.float32)
      + bl1_ref[0:1, :])
  logits = (jnp.dot(z, wl2_ref[...], preferred_element_type=jnp.float32)
            + bl2_ref[0:1, :])
  col = lax.broadcasted_iota(jnp.int32, (NUM_GRAPHS, 128), 1)
  lm = jnp.where(col < NUM_CLASSES, logits, -1e30)
  m = jnp.max(lm, axis=1, keepdims=True)
  s = jnp.sum(jnp.exp(lm - m), axis=1, keepdims=True)
  o_ref[...] = lm - m - jnp.log(s)


def _head(pf, wc, bc8, wl1p, bl18, wl2p, bl28):
  return pl.pallas_call(
      _head_body,
      out_shape=jax.ShapeDtypeStruct((NUM_GRAPHS, 128), jnp.float32),
  )(pf, wc, bc8, wl1p, bl18, wl2p, bl28)


def kernel(x, edge_index, batch, W1, b1, W2, b2, W3, b3, Wc, bc,
           Wl1, bl1, Wl2, bl2):
  n = x.shape[0]
  e = edge_index.shape[1]
  epad = ((e + NWORK * CHUNK - 1) // (NWORK * CHUNK)) * (NWORK * CHUNK)

  src = jnp.concatenate([edge_index[0].astype(jnp.int32),
                         jnp.full((epad - e,), ZROW, jnp.int32)])
  dst = jnp.concatenate([edge_index[1].astype(jnp.int32),
                         jnp.full((epad - e,), ZROW, jnp.int32)])
  xpad = jnp.zeros((NPAD, DIM), jnp.float32).at[:n].set(x)
  batch_pad = jnp.concatenate([batch.astype(jnp.int32),
                               jnp.full((NPAD - n,), NUM_GRAPHS, jnp.int32)])
  batch8 = jnp.broadcast_to(batch_pad[:, None], (NPAD, 8))

  deg_kernel = _make_deg_kernel(epad)
  agg_kernel = _make_agg_kernel(epad)

  deg_parts = deg_kernel(dst)
  pc = jnp.zeros((NPAD, 8), jnp.float32).at[:, :2].set(deg_parts[:, :, 0].T)

  def pad_b(b):
    return jnp.broadcast_to(b[None, :], (8, DIM))

  hs = _t0(xpad, pc, W1)
  for (bl, wn) in ((b1, W2), (b2, W3)):
    parts = agg_kernel(hs, src, dst)
    hs = _tmid(parts[0], parts[1], hs, pc, pad_b(bl), wn)
  parts = agg_kernel(hs, src, dst)
  h3 = _tlast(parts[0], parts[1], hs, pc, pad_b(b3))

  keys8 = h3[:, DIM - 8:DIM]
  sel = _topk(keys8, batch8)
  idx_flat = sel[:TOPK].T.reshape(NUM_GRAPHS * TOPK)

  row_gather = _make_row_gather_kernel(NUM_GRAPHS * TOPK)
  rows = row_gather(h3, idx_flat)
  pf = rows.reshape(NUM_GRAPHS, TOPK * DIM)

  wc_flat = Wc.transpose(2, 1, 0).reshape(DIM * KSIZE, CONV_OUT)
  wl1p = Wl1.reshape(CONV_OUT, _CONV_T, DIM).transpose(1, 0, 2).reshape(
      CONV_OUT * _CONV_T, DIM)
  wl2p = jnp.zeros((DIM, 128), jnp.float32).at[:, :NUM_CLASSES].set(Wl2)
  bl2p = jnp.zeros((128,), jnp.float32).at[:NUM_CLASSES].set(bl2)
  bc_pad = jnp.broadcast_to(bc[None, :], (8, CONV_OUT))

  out = _head(pf, wc_flat, bc_pad, wl1p, pad_b(bl1), wl2p, pad_b(bl2p))
  return out[:, :NUM_CLASSES]
